# Initial kernel scaffold; baseline (speedup 1.0000x reference)
import jax
import jax.numpy as jnp
from jax.experimental import pallas as pl


def kernel(positions, table):
    raise NotImplementedError("write your pallas kernel here")


# speedup vs baseline: 3.9565x; 3.9565x over previous
import functools

import jax
import jax.numpy as jnp
from jax import lax
from jax.experimental import pallas as pl
from jax.experimental.pallas import tpu as pltpu
from jax.experimental.pallas import tpu_sc as plsc

EMBED = 64
NC = 2
NS = 16
NW = NC * NS

CHUNK = 512
SUB = 128
NSUB = CHUNK // SUB


def _make_sc_gather(B):
    PW = B // NW
    G = PW // CHUNK

    mesh = plsc.VectorSubcoreMesh(core_axis_name="c", subcore_axis_name="s")

    @functools.partial(
        pl.kernel,
        mesh=mesh,
        out_type=jax.ShapeDtypeStruct((B, EMBED), jnp.float32),
        scratch_types=[
            pltpu.VMEM((NSUB, SUB), jnp.int32),
            pltpu.VMEM((CHUNK, EMBED), jnp.float32),
            pltpu.SemaphoreType.DMA,
        ],
        compiler_params=pltpu.CompilerParams(use_tc_tiling_on_sc=False),
    )
    def k(idx_hbm, table_hbm, out_hbm, idx_v, rows_v, sem):
        wid = lax.axis_index("s") * NC + lax.axis_index("c")

        def body(g, carry):
            base = wid * PW + g * CHUNK
            pltpu.sync_copy(idx_hbm.at[wid, g], idx_v)
            for j in range(NSUB):
                pltpu.async_copy(
                    table_hbm.at[idx_v.at[j]],
                    rows_v.at[pl.ds(j * SUB, SUB)],
                    sem,
                ).wait()
            pltpu.sync_copy(rows_v, out_hbm.at[pl.ds(base, CHUNK)])
            return carry

        lax.fori_loop(0, G, body, 0)

    return k


def kernel(positions, table):
    batch, seq = positions.shape
    B = batch * seq
    idx = positions.reshape(NW, B // (NW * CHUNK), NSUB, SUB).astype(jnp.int32)
    out = _make_sc_gather(B)(idx, table)
    return out.reshape(batch, seq, EMBED)


# speedup vs baseline: 4.5323x; 1.1455x over previous
import functools

import jax
import jax.numpy as jnp
from jax import lax
from jax.experimental import pallas as pl
from jax.experimental.pallas import tpu as pltpu
from jax.experimental.pallas import tpu_sc as plsc

EMBED = 64
NC = 2
NS = 16
NW = NC * NS

SUB = 128
IDXROWS = 8
GROUP = IDXROWS * SUB
CHUNK = 256
NSUB = CHUNK // SUB
QPG = GROUP // CHUNK
LANES = 16


def _make_sc_gather(B):
    PW = B // NW
    G = PW // GROUP

    mesh = plsc.VectorSubcoreMesh(core_axis_name="c", subcore_axis_name="s")

    @functools.partial(
        pl.kernel,
        mesh=mesh,
        out_type=jax.ShapeDtypeStruct((B, EMBED), jnp.float32),
        scratch_types=[
            pltpu.VMEM((IDXROWS, SUB), jnp.int32),
            pltpu.VMEM((CHUNK, 2 * EMBED), jnp.float32),
            pltpu.VMEM((CHUNK, EMBED), jnp.float32),
            pltpu.SemaphoreType.DMA,
        ],
    )
    def k(idx_hbm, table_hbm, out_hbm, idx_v, rows_v, rows_t, sem):
        wid = lax.axis_index("s") * NC + lax.axis_index("c")

        def body(g, carry):
            pltpu.sync_copy(idx_hbm.at[wid, g], idx_v)
            for q in range(QPG):
                base = wid * PW + g * GROUP + q * CHUNK
                for j in range(NSUB):
                    pltpu.async_copy(
                        table_hbm.at[idx_v.at[q * NSUB + j]],
                        rows_v.at[pl.ds(j * SUB, SUB)],
                        sem,
                    ).wait()

                def repack(r, c2):
                    for c in range(EMBED // LANES):
                        rows_t[r, pl.ds(c * LANES, LANES)] = (
                            rows_v[r, pl.ds(c * LANES, LANES)])
                    return c2

                lax.fori_loop(0, CHUNK, repack, 0)
                pltpu.sync_copy(rows_t, out_hbm.at[pl.ds(base, CHUNK)])
            return carry

        lax.fori_loop(0, G, body, 0)

    return k


def kernel(positions, table):
    batch, seq = positions.shape
    B = batch * seq
    idx = positions.reshape(NW, B // (NW * GROUP), IDXROWS, SUB).astype(jnp.int32)
    table_pad = jnp.pad(table, ((0, 0), (0, EMBED)))
    out = _make_sc_gather(B)(idx, table_pad)
    return out.reshape(batch, seq, EMBED)
